# baseline (speedup 1.0000x reference)
import jax
import jax.numpy as jnp
from jax import lax
from jax.experimental import pallas as pl
from jax.experimental.pallas import tpu as pltpu
from jax.experimental.pallas import tpu_sc as plsc

V = 1000000
D = 64
B = 16384
L = 20
N = B * L

NC = 2
NS = 16
NW = NC * NS

NB = 4
CB = NB * L
B_PER_W = B // NW
GCHUNKS = B_PER_W // NB
N_PER_W = N // NW

KB = 1600
KCHUNKS = V // KB


def _repack_body(w_hbm, out_hbm, in_v, out_v, sem):
  wid = lax.axis_index("s") * NC + lax.axis_index("c")
  lanes = lax.iota(jnp.int32, 16)
  nmine = (KCHUNKS - wid + NW - 1) // NW

  cols = [jnp.full((16,), c, jnp.int32) for c in range(16)]
  m00ff = jnp.full((16,), 0x00FF00FF, jnp.int32)
  mff00 = jnp.full((16,), 0xFF00FF00 - (1 << 32), jnp.int32)
  mlo16 = jnp.full((16,), 0xFFFF, jnp.int32)
  mhi16 = jnp.full((16,), 0xFFFF0000 - (1 << 32), jnp.int32)

  def chunk(ci, _):
    r0 = (ci * NW + wid) * KB
    pltpu.sync_copy(w_hbm.at[:, pl.ds(pl.multiple_of(r0, 32), KB)], in_v)

    def kblock(kb, _):
      k0 = 64 * kb
      rowbase = k0 + 4 * lanes
      for cg in range(16):
        x = [plsc.bitcast(
            in_v[4 * cg + b, pl.ds(pl.multiple_of(k0, 64), 64)], jnp.int32)
            for b in range(4)]
        ab_lo = (x[0] & m00ff) | ((x[1] << 8) & mff00)
        ab_hi = ((x[0] >> 8) & m00ff) | (x[1] & mff00)
        cd_lo = (x[2] & m00ff) | ((x[3] << 8) & mff00)
        cd_hi = ((x[2] >> 8) & m00ff) | (x[3] & mff00)
        w0 = (ab_lo & mlo16) | (cd_lo << 16)
        w1 = (ab_hi & mlo16) | (cd_hi << 16)
        w2 = ((ab_lo >> 16) & mlo16) | (cd_lo & mhi16)
        w3 = ((ab_hi >> 16) & mlo16) | (cd_hi & mhi16)
        for p, w in enumerate((w0, w1, w2, w3)):
          plsc.store_scatter(out_v, [rowbase + p, cols[cg]], w)
      return 0

    lax.fori_loop(0, KB // 64, kblock, 0)
    pltpu.sync_copy(out_v, out_hbm.at[pl.ds(pl.multiple_of(r0, 32), KB)])
    return 0

  lax.fori_loop(0, nmine, chunk, 0)


def _gather_body(idx_hbm, w_hbm, s_hbm, m_hbm, out_hbm, idx_v, rows_v, s_v,
                 m_v, stage_v, obuf_v, sem):
  wid = lax.axis_index("s") * NC + lax.axis_index("c")
  lanes = lax.iota(jnp.int32, 16)
  zeros16 = jnp.full((16,), 0, jnp.int32)

  def chunk_body(ci, _):
    b0 = wid * B_PER_W + ci * NB
    base = b0 * L
    pltpu.sync_copy(idx_hbm.at[pl.ds(base, CB)], idx_v)
    cw = pltpu.async_copy(w_hbm.at[idx_v], rows_v, sem)
    cs = pltpu.async_copy(s_hbm.at[idx_v], s_v, sem)
    cm = pltpu.async_copy(m_hbm.at[idx_v], m_v, sem)
    cw.wait()
    cs.wait()
    cm.wait()

    def row_body(r, _):
      w32 = rows_v[r]
      rsplat = zeros16 + r
      sv = plsc.load_gather(s_v, [rsplat, zeros16])
      mv = plsc.load_gather(m_v, [rsplat, zeros16])
      o0 = sv * (((w32 << 24) >> 24).astype(jnp.float32) + mv)
      o1 = sv * (((w32 << 16) >> 24).astype(jnp.float32) + mv)
      o2 = sv * (((w32 << 8) >> 24).astype(jnp.float32) + mv)
      o3 = sv * ((w32 >> 24).astype(jnp.float32) + mv)
      p01 = plsc.bitcast(
          plsc.pack(o0, o1, format=plsc.PackFormat.INTERLEAVED), jnp.int32)
      p23 = plsc.bitcast(
          plsc.pack(o2, o3, format=plsc.PackFormat.INTERLEAVED), jnp.int32)
      wbase = r * (D // 2) + 2 * lanes
      plsc.store_scatter(stage_v, [wbase], p01)
      plsc.store_scatter(stage_v, [wbase + 1], p23)
      return 0

    lax.fori_loop(0, CB, row_body, 0)

    def reorder_body(bb, _):
      for l in range(L):
        r = bb * L + l
        half0 = plsc.bitcast(
            stage_v[pl.ds(pl.multiple_of(r * (D // 2), 32), 16)],
            jnp.bfloat16)
        half1 = plsc.bitcast(
            stage_v[pl.ds(pl.multiple_of(r * (D // 2), 32) + 16, 16)],
            jnp.bfloat16)
        obuf_v[bb, l, pl.ds(0, 32)] = half0
        obuf_v[bb, l, pl.ds(32, 32)] = half1
      return 0

    lax.fori_loop(0, NB, reorder_body, 0)
    pltpu.sync_copy(obuf_v, out_hbm.at[pl.ds(b0, NB)])
    return 0

  lax.fori_loop(0, GCHUNKS, chunk_body, 0)


@jax.jit
def kernel(idx, weight, scales, means):
  mesh = plsc.VectorSubcoreMesh(core_axis_name="c", subcore_axis_name="s")
  params = pltpu.CompilerParams(
      needs_layout_passes=False, use_tc_tiling_on_sc=False)
  repack = pl.kernel(
      _repack_body,
      out_type=jax.ShapeDtypeStruct((V, D // 4), jnp.int32),
      mesh=mesh,
      compiler_params=params,
      scratch_types=[
          pltpu.VMEM((D, KB), jnp.int8),
          pltpu.VMEM((KB, D // 4), jnp.int32),
          pltpu.SemaphoreType.DMA,
      ],
  )
  gather = pl.kernel(
      _gather_body,
      out_type=jax.ShapeDtypeStruct((B, L, D), jnp.bfloat16),
      mesh=mesh,
      compiler_params=params,
      scratch_types=[
          pltpu.VMEM((CB,), jnp.int32),
          pltpu.VMEM((CB, D // 4), jnp.int32),
          pltpu.VMEM((CB, 1), jnp.float32),
          pltpu.VMEM((CB, 1), jnp.float32),
          pltpu.VMEM((CB * D // 2,), jnp.int32),
          pltpu.VMEM((NB, L, D), jnp.bfloat16),
          pltpu.SemaphoreType.DMA,
      ],
  )
  w32 = repack(weight.T)
  out = gather(idx.reshape(N), w32, scales, means)
  return out


# speedup vs baseline: 7.9348x; 7.9348x over previous
import jax
import jax.numpy as jnp
from jax import lax
from jax.experimental import pallas as pl
from jax.experimental.pallas import tpu as pltpu
from jax.experimental.pallas import tpu_sc as plsc

V = 1000000
D = 64
B = 16384
L = 20
N = B * L

NC = 2
NS = 16
NW = NC * NS

CB = 128
N_PER_W = N // NW
GCHUNKS = N_PER_W // CB

KB = 1250
V_PER_W = V // NW
KCHUNKS = V_PER_W // KB


def _repack_body(w_hbm, out_hbm, in_v, out_v, sem_i, sem_o):
  wid = lax.axis_index("s") * NC + lax.axis_index("c")
  base = wid * V_PER_W
  pltpu.async_copy(w_hbm.at[pl.ds(base, KB)], in_v.at[0], sem_i)

  def chunk(ci, _):
    slot = ci % 2
    r0 = base + ci * KB
    pltpu.make_async_copy(w_hbm.at[pl.ds(r0, KB)], in_v.at[slot],
                          sem_i).wait()

    @pl.when(ci + 1 < KCHUNKS)
    def _prefetch():
      pltpu.async_copy(w_hbm.at[pl.ds(r0 + KB, KB)], in_v.at[1 - slot],
                       sem_i)

    @pl.when(ci >= 1)
    def _drain():
      pltpu.make_async_copy(out_v.at[1 - slot],
                            out_hbm.at[pl.ds(r0 - KB, KB)], sem_o).wait()

    def row(q, _):
      for j in range(2):
        out_v[slot, 2 * q + j] = plsc.bitcast(in_v[slot, 2 * q + j],
                                              jnp.int32)
      return 0

    lax.fori_loop(0, KB // 2, row, 0)
    pltpu.async_copy(out_v.at[slot], out_hbm.at[pl.ds(r0, KB)], sem_o)
    return 0

  lax.fori_loop(0, KCHUNKS, chunk, 0)
  pltpu.make_async_copy(out_v.at[(KCHUNKS - 1) % 2],
                        out_hbm.at[pl.ds(base + (KCHUNKS - 1) * KB, KB)],
                        sem_o).wait()


def _issue_gathers(idx_hbm, w_hbm, s_hbm, m_hbm, idx_v, rows_v, s_v, m_v,
                   slot, base, sem_i, sem_g):
  pltpu.make_async_copy(idx_hbm.at[pl.ds(base, CB)], idx_v.at[slot],
                        sem_i).wait()
  pltpu.async_copy(w_hbm.at[idx_v.at[slot]], rows_v.at[slot], sem_g)
  pltpu.async_copy(s_hbm.at[idx_v.at[slot]], s_v.at[slot], sem_g)
  pltpu.async_copy(m_hbm.at[idx_v.at[slot]], m_v.at[slot], sem_g)


def _gather_body(idx_hbm, w_hbm, s_hbm, m_hbm, out_hbm, idx_v, rows_v, s_v,
                 m_v, stage_v, obuf_v, sem_i, sem_g, sem_o):
  wid = lax.axis_index("s") * NC + lax.axis_index("c")
  lanes = lax.iota(jnp.int32, 16)
  zeros16 = jnp.full((16,), 0, jnp.int32)
  wbase = wid * N_PER_W

  pltpu.async_copy(idx_hbm.at[pl.ds(wbase, CB)], idx_v.at[0], sem_i)
  _issue_gathers(idx_hbm, w_hbm, s_hbm, m_hbm, idx_v, rows_v, s_v, m_v, 0,
                 wbase, sem_i, sem_g)

  def chunk_body(ci, _):
    slot = ci % 2
    base = wbase + ci * CB

    pltpu.make_async_copy(w_hbm.at[idx_v.at[slot]], rows_v.at[slot],
                          sem_g).wait()
    pltpu.make_async_copy(s_hbm.at[idx_v.at[slot]], s_v.at[slot],
                          sem_g).wait()
    pltpu.make_async_copy(m_hbm.at[idx_v.at[slot]], m_v.at[slot],
                          sem_g).wait()

    @pl.when(ci + 1 < GCHUNKS)
    def _prefetch():
      nbase = base + CB
      pltpu.async_copy(idx_hbm.at[pl.ds(nbase, CB)], idx_v.at[1 - slot],
                       sem_i)
      _issue_gathers(idx_hbm, w_hbm, s_hbm, m_hbm, idx_v, rows_v, s_v, m_v,
                     1 - slot, nbase, sem_i, sem_g)

    @pl.when(ci >= 1)
    def _drain():
      pltpu.make_async_copy(obuf_v.at[1 - slot],
                            out_hbm.at[pl.ds(base - CB, CB)], sem_o).wait()

    def row_body(r, _):
      w32 = rows_v[slot, r]
      rsplat = zeros16 + r
      sv = plsc.load_gather(s_v.at[slot], [rsplat])
      mv = plsc.load_gather(m_v.at[slot], [rsplat])
      o0 = sv * (((w32 << 24) >> 24).astype(jnp.float32) + mv)
      o1 = sv * (((w32 << 16) >> 24).astype(jnp.float32) + mv)
      o2 = sv * (((w32 << 8) >> 24).astype(jnp.float32) + mv)
      o3 = sv * ((w32 >> 24).astype(jnp.float32) + mv)
      p01 = plsc.bitcast(
          plsc.pack(o0, o1, format=plsc.PackFormat.INTERLEAVED), jnp.int32)
      p23 = plsc.bitcast(
          plsc.pack(o2, o3, format=plsc.PackFormat.INTERLEAVED), jnp.int32)
      sbase = r * (D // 2) + 2 * lanes
      plsc.store_scatter(stage_v, [sbase], p01)
      plsc.store_scatter(stage_v, [sbase + 1], p23)
      return 0

    lax.fori_loop(0, CB, row_body, 0)

    def reorder_body(t, _):
      for j in range(2):
        r = 2 * t + j
        off = pl.multiple_of(r * (D // 2), 32)
        obuf_v[slot, r, pl.ds(0, 32)] = plsc.bitcast(
            stage_v[pl.ds(off, 16)], jnp.bfloat16)
        obuf_v[slot, r, pl.ds(32, 32)] = plsc.bitcast(
            stage_v[pl.ds(off + 16, 16)], jnp.bfloat16)
      return 0

    lax.fori_loop(0, CB // 2, reorder_body, 0)
    pltpu.async_copy(obuf_v.at[slot], out_hbm.at[pl.ds(base, CB)], sem_o)
    return 0

  lax.fori_loop(0, GCHUNKS, chunk_body, 0)
  pltpu.make_async_copy(obuf_v.at[(GCHUNKS - 1) % 2],
                        out_hbm.at[pl.ds(wbase + (GCHUNKS - 1) * CB, CB)],
                        sem_o).wait()


@jax.jit
def kernel(idx, weight, scales, means):
  mesh = plsc.VectorSubcoreMesh(core_axis_name="c", subcore_axis_name="s")
  params = pltpu.CompilerParams(
      needs_layout_passes=False, use_tc_tiling_on_sc=False)
  repack = pl.kernel(
      _repack_body,
      out_type=jax.ShapeDtypeStruct((V, D // 4), jnp.int32),
      mesh=mesh,
      compiler_params=params,
      scratch_types=[
          pltpu.VMEM((2, KB, D), jnp.int8),
          pltpu.VMEM((2, KB, D // 4), jnp.int32),
          pltpu.SemaphoreType.DMA,
          pltpu.SemaphoreType.DMA,
      ],
  )
  gather = pl.kernel(
      _gather_body,
      out_type=jax.ShapeDtypeStruct((N, D), jnp.bfloat16),
      mesh=mesh,
      compiler_params=params,
      scratch_types=[
          pltpu.VMEM((2, CB), jnp.int32),
          pltpu.VMEM((2, CB, D // 4), jnp.int32),
          pltpu.VMEM((2, CB), jnp.float32),
          pltpu.VMEM((2, CB), jnp.float32),
          pltpu.VMEM((CB * D // 2,), jnp.int32),
          pltpu.VMEM((2, CB, D), jnp.bfloat16),
          pltpu.SemaphoreType.DMA,
          pltpu.SemaphoreType.DMA,
          pltpu.SemaphoreType.DMA,
      ],
  )
  w32 = repack(weight)
  out = gather(idx.reshape(N), w32, scales.reshape(V), means.reshape(V))
  return out.reshape(B, L, D)
